# Initial kernel scaffold; baseline (speedup 1.0000x reference)
import jax
import jax.numpy as jnp
from jax.experimental import pallas as pl


def kernel(x, edge_index, edge_attr, batch, demographics, emb, Wrel1, brel1, Wroot1, Wrel2, brel2, Wroot2, Wrel3, brel3, Wroot3, Wc1, bc1, Wc2, bc2):
    raise NotImplementedError("write your pallas kernel here")


# speedup vs baseline: 9.1312x; 9.1312x over previous
import functools

import jax
import jax.numpy as jnp
from jax import lax
from jax.experimental import pallas as pl
from jax.experimental.pallas import tpu as pltpu
from jax.experimental.pallas import tpu_sc as plsc

N = 100000
E = 1600000
NG = 16
F = 16
NC = 2
NS = 16
NW = NC * NS
CH = 128
CPW = 392
EPW = CPW * CH
E_PAD = NW * EPW
N_PAD = 100352
NPS = N_PAD // NS
SB = 49
NSB = CPW // SB
BLK = 2048
GRID = N_PAD // BLK

@functools.cache
def _mesh():
    return plsc.VectorSubcoreMesh(core_axis_name="c", subcore_axis_name="s",
                                  num_cores=NC, num_subcores=NS)


def _sc_gather_rows(table, idx):
    npw = N_PAD // NW
    c_sz = 64
    n_ch = npw // c_sz

    @functools.partial(
        pl.kernel,
        mesh=_mesh(),
        out_type=jax.ShapeDtypeStruct((N_PAD, F), jnp.float32),
        compiler_params=pltpu.CompilerParams(use_tc_tiling_on_sc=False),
        scratch_types=[
            pltpu.VMEM((npw,), jnp.int32),
            pltpu.VMEM((c_sz, F), jnp.float32),
            pltpu.SemaphoreType.DMA,
        ],
    )
    def k(table_hbm, idx_hbm, out_hbm, idx_v, rows_v, sem):
        wid = lax.axis_index("s") * NC + lax.axis_index("c")
        base = wid * npw
        pltpu.sync_copy(idx_hbm.at[pl.ds(base, npw)], idx_v)

        def body(c, carry):
            off = c * c_sz
            pltpu.async_copy(table_hbm.at[idx_v.at[pl.ds(off, c_sz)]], rows_v, sem).wait()
            pltpu.sync_copy(rows_v, out_hbm.at[pl.ds(base + off, c_sz), :])
            return carry

        lax.fori_loop(0, n_ch, body, 0)

    return k(table, idx)


def _sc_edge_pass(hprev, srcp, dstp, wp):
    scratch = [
        pltpu.VMEM((SB * CH,), jnp.int32),
        pltpu.VMEM((SB * CH,), jnp.int32),
        pltpu.VMEM((SB * CH,), jnp.float32),
        pltpu.VMEM((CH, F), jnp.float32),
        pltpu.VMEM((CH, F), jnp.float32),
        pltpu.VMEM_SHARED((N_PAD, F), jnp.float32),
        pltpu.SemaphoreType.DMA,
    ]

    @functools.partial(pl.kernel, mesh=_mesh(),
                       out_type=jax.ShapeDtypeStruct((NC, N_PAD, F),
                                                     jnp.float32),
                       compiler_params=pltpu.CompilerParams(
                           use_tc_tiling_on_sc=False),
                       scratch_types=scratch)
    def k(h_hbm, src_hbm, dst_hbm, w_hbm, bh_hbm,
          src_v, dst_v, w_v, rows_v, zf_v, acc_sh, sem):
        core = lax.axis_index("c")
        sub = lax.axis_index("s")
        wid = sub * NC + core

        zero16 = jnp.zeros((16,), jnp.float32)
        for j in range(CH):
            zf_v[j] = zero16
        def zbody(c, carry):
            pltpu.sync_copy(zf_v, acc_sh.at[pl.ds(sub * NPS + c * CH, CH), :])
            return carry
        lax.fori_loop(0, NPS // CH, zbody, 0)
        plsc.subcore_barrier()

        ebase = wid * EPW

        def sb_body(s, carry):
            sb0 = ebase + s * (SB * CH)
            pltpu.sync_copy(src_hbm.at[pl.ds(sb0, SB * CH)], src_v)
            pltpu.sync_copy(w_hbm.at[pl.ds(sb0, SB * CH)], w_v)
            pltpu.sync_copy(dst_hbm.at[pl.ds(sb0, SB * CH)], dst_v)

            def ch_body(c, carry2):
                off = c * CH
                idx = src_v.at[pl.ds(off, CH)]
                pltpu.async_copy(h_hbm.at[idx], rows_v, sem).wait()
                for g in range(CH // 16):
                    wv = w_v[pl.ds(off + g * 16, 16)]
                    for l in range(16):
                        j = g * 16 + l
                        rows_v[j] = rows_v[j] * wv[l]
                dix = dst_v.at[pl.ds(off, CH)]
                pltpu.sync_copy(rows_v, acc_sh.at[dix], add=True)
                return carry2

            lax.fori_loop(0, SB, ch_body, 0)
            return carry

        lax.fori_loop(0, NSB, sb_body, 0)

        plsc.subcore_barrier()
        r0 = sub * NPS
        pltpu.sync_copy(acc_sh.at[pl.ds(r0, NPS), :],
                        bh_hbm.at[core, pl.ds(r0, NPS), :])

    return k(hprev, srcp, dstp, wp)


def _sc_d_pass(srcp, dstp, wp, dvec):
    want_d2 = dvec is not None
    scratch = [
        pltpu.VMEM((SB * CH,), jnp.int32),
        pltpu.VMEM((SB * CH,), jnp.int32),
        pltpu.VMEM((SB * CH,), jnp.float32),
        pltpu.VMEM((CH,), jnp.float32),
        pltpu.VMEM((CH,), jnp.float32),
        pltpu.VMEM((NPS,), jnp.float32),
        pltpu.VMEM_SHARED((N_PAD,), jnp.float32),
        pltpu.SemaphoreType.DMA,
    ]

    @functools.partial(pl.kernel, mesh=_mesh(),
                       out_type=jax.ShapeDtypeStruct((NC, N_PAD), jnp.float32),
                       compiler_params=pltpu.CompilerParams(
                           use_tc_tiling_on_sc=False),
                       scratch_types=scratch)
    def k(src_hbm, dst_hbm, w_hbm, d_hbm, bd_hbm,
          src_v, dst_v, w_v, dsrc_v, d2_v, zd_v, accd_sh, sem):
        core = lax.axis_index("c")
        sub = lax.axis_index("s")
        wid = sub * NC + core

        zero16 = jnp.zeros((16,), jnp.float32)
        for j in range(NPS // 16):
            zd_v[pl.ds(j * 16, 16)] = zero16
        pltpu.sync_copy(zd_v, accd_sh.at[pl.ds(sub * NPS, NPS)])
        plsc.subcore_barrier()

        ebase = wid * EPW

        def sb_body(s, carry):
            sb0 = ebase + s * (SB * CH)
            if want_d2:
                pltpu.sync_copy(src_hbm.at[pl.ds(sb0, SB * CH)], src_v)
            pltpu.sync_copy(w_hbm.at[pl.ds(sb0, SB * CH)], w_v)
            pltpu.sync_copy(dst_hbm.at[pl.ds(sb0, SB * CH)], dst_v)

            def ch_body(c, carry2):
                off = c * CH
                dix = dst_v.at[pl.ds(off, CH)]
                if want_d2:
                    idx = src_v.at[pl.ds(off, CH)]
                    pltpu.async_copy(d_hbm.at[idx], dsrc_v, sem).wait()
                    for g in range(CH // 16):
                        d2_v[pl.ds(g * 16, 16)] = (
                            dsrc_v[pl.ds(g * 16, 16)]
                            * w_v[pl.ds(off + g * 16, 16)])
                    pltpu.sync_copy(d2_v, accd_sh.at[dix], add=True)
                else:
                    pltpu.sync_copy(w_v.at[pl.ds(off, CH)],
                                    accd_sh.at[dix], add=True)
                return carry2

            lax.fori_loop(0, SB, ch_body, 0)
            return carry

        lax.fori_loop(0, NSB, sb_body, 0)

        plsc.subcore_barrier()
        r0 = sub * NPS
        pltpu.sync_copy(accd_sh.at[pl.ds(r0, NPS)],
                        bd_hbm.at[core, pl.ds(r0, NPS)])

    return k(srcp, dstp, wp,
             dvec if want_d2 else jnp.zeros((N_PAD,), jnp.float32))


def _tc_combine(bh0, bh1, bd0, bd1, extra, batchp, want_h, want_d, want_extra):
    outs = []
    if want_h:
        outs.append(jax.ShapeDtypeStruct((N_PAD, F), jnp.float32))
    outs.append(jax.ShapeDtypeStruct((NG, F), jnp.float32))
    if want_d:
        outs.append(jax.ShapeDtypeStruct((N_PAD,), jnp.float32))
    if want_d:
        outs.append(jax.ShapeDtypeStruct((NG, 1), jnp.float32))
    if want_extra:
        outs.append(jax.ShapeDtypeStruct((NG, F), jnp.float32))
    outs.append(jax.ShapeDtypeStruct((NG, 1), jnp.float32))

    def body(*refs):
        i = pl.program_id(0)
        it = iter(refs)
        bh0_r = next(it); bh1_r = next(it)
        bd0_r = next(it) if want_d else None
        bd1_r = next(it) if want_d else None
        ex_r = next(it) if want_extra else None
        bt_r = next(it)
        h_r = next(it) if want_h else None
        ph_r = next(it)
        d_r = next(it) if want_d else None
        sd_r = next(it) if want_d else None
        pe_r = next(it) if want_extra else None
        cnt_r = next(it)
        ph_s = next(it)
        sd_s = next(it) if want_d else None
        pe_s = next(it) if want_extra else None
        cnt_s = next(it)

        @pl.when(i == 0)
        def _init():
            ph_s[...] = jnp.zeros((NG, F), jnp.float32)
            cnt_s[...] = jnp.zeros((NG, 1), jnp.float32)
            if want_d:
                sd_s[...] = jnp.zeros((NG, 1), jnp.float32)
            if want_extra:
                pe_s[...] = jnp.zeros((NG, F), jnp.float32)

        hb = bh0_r[...] + bh1_r[...]
        if want_h:
            h_r[...] = hb
        bt = bt_r[...]
        oh = (bt[:, None] == lax.broadcasted_iota(jnp.int32, (1, NG), 1)
              ).astype(jnp.float32)
        dn = (((0,), (0,)), ((), ()))
        ph_s[...] += lax.dot_general(oh, hb, dn,
                                     preferred_element_type=jnp.float32)
        cnt_s[...] += lax.dot_general(oh, jnp.ones((BLK, 1), jnp.float32), dn,
                                      preferred_element_type=jnp.float32)
        if want_d:
            db = bd0_r[...] + bd1_r[...]
            d_r[...] = db
            sd_s[...] += lax.dot_general(oh, db[:, None], dn,
                                         preferred_element_type=jnp.float32)
        if want_extra:
            pe_s[...] += lax.dot_general(oh, ex_r[...], dn,
                                         preferred_element_type=jnp.float32)

        @pl.when(i == GRID - 1)
        def _fin():
            ph_r[...] = ph_s[...]
            cnt_r[...] = cnt_s[...]
            if want_d:
                sd_r[...] = sd_s[...]
            if want_extra:
                pe_r[...] = pe_s[...]

    row = pl.BlockSpec((BLK, F), lambda i: (i, 0))
    vec = pl.BlockSpec((BLK,), lambda i: (i,))
    fixg = pl.BlockSpec((NG, F), lambda i: (0, 0))
    fix1 = pl.BlockSpec((NG, 1), lambda i: (0, 0))

    in_specs = [row, row]
    inputs = [bh0, bh1]
    if want_d:
        in_specs += [vec, vec]
        inputs += [bd0, bd1]
    if want_extra:
        in_specs += [row]
        inputs += [extra]
    in_specs += [vec]
    inputs += [batchp]

    out_specs = []
    if want_h:
        out_specs.append(row)
    out_specs.append(fixg)
    if want_d:
        out_specs += [vec, fix1]
    if want_extra:
        out_specs.append(fixg)
    out_specs.append(fix1)

    scratch = [pltpu.VMEM((NG, F), jnp.float32)]
    if want_d:
        scratch.append(pltpu.VMEM((NG, 1), jnp.float32))
    if want_extra:
        scratch.append(pltpu.VMEM((NG, F), jnp.float32))
    scratch.append(pltpu.VMEM((NG, 1), jnp.float32))

    return pl.pallas_call(
        body,
        grid=(GRID,),
        in_specs=in_specs,
        out_specs=tuple(out_specs),
        out_shape=tuple(outs),
        scratch_shapes=scratch,
    )(*inputs)


def _tc_head(P0, P1, P2, P3, sd, sd2, cnt, demo,
             Wrel1, brel1, Wroot1, Wrel2, brel2, Wroot2, Wrel3, brel3, Wroot3,
             Wc1a, Wc1b, bc1, Wc2, bc2):
    def body(p0, p1, p2, p3, sd_r, sd2_r, cnt_r, demo_r,
             wr1, br1, wo1, wr2, br2, wo2, wr3, br3, wo3,
             wc1a, wc1b, bc1_r, wc2, bc2_r, out_r):
        dnT = (((1,), (1,)), ((), ()))
        mm = lambda a, b: jnp.dot(a, b, preferred_element_type=jnp.float32)
        rmm = lambda a, b: lax.dot_general(a, b, dnT,
                                           preferred_element_type=jnp.float32)
        wr1a, wo1a = wr1[...], wo1[...]
        wr2a, wo2a = wr2[...], wo2[...]
        wr3a, wo3a = wr3[...], wo3[...]
        wo21 = mm(wo2a, wo1a)
        wr2o1 = mm(wr2a, wo1a)
        wo2r1 = mm(wo2a, wr1a)
        wr21 = mm(wr2a, wr1a)
        C0T = mm(wo3a, wo21)
        C1T = mm(wo3a, wr2o1) + mm(wo3a, wo2r1) + mm(wr3a, wo21)
        C2T = mm(wo3a, wr21) + mm(wr3a, wr2o1) + mm(wr3a, wo2r1)
        C3T = mm(wr3a, wr21)
        b1r, b2r, b3r = br1[...], br2[...], br3[...]
        g0r = rmm(rmm(b1r, wo2a), wo3a) + rmm(b2r, wo3a) + b3r
        g1r = rmm(rmm(b1r, wr2a), wo3a) + rmm(rmm(b1r, wo2a), wr3a) \
            + rmm(b2r, wr3a)
        g2r = rmm(rmm(b1r, wr2a), wr3a)

        cnt = cnt_r[...]
        inv = 1.0 / jnp.maximum(cnt, 1.0)
        has = (cnt > 0.0).astype(jnp.float32)
        pooled = (
            rmm(p0[...] * inv, C0T) + rmm(p1[...] * inv, C1T)
            + rmm(p2[...] * inv, C2T) + rmm(p3[...] * inv, C3T)
            + has * g0r + (sd_r[...] * inv) * g1r
            + (sd2_r[...] * inv) * g2r)
        z = (rmm(pooled, wc1a[...]) + rmm(demo_r[...], wc1b[...])
             + bc1_r[...])
        z = jnp.maximum(z, 0.0)
        out_r[...] = rmm(z, wc2[...]) + bc2_r[...]

    return pl.pallas_call(
        body,
        out_shape=jax.ShapeDtypeStruct((NG, 10), jnp.float32),
    )(P0, P1, P2, P3, sd, sd2, cnt, demo,
      Wrel1, brel1[None, :], Wroot1, Wrel2, brel2[None, :], Wroot2,
      Wrel3, brel3[None, :], Wroot3,
      Wc1a, Wc1b, bc1[None, :], Wc2, bc2[None, :])


def kernel(x, edge_index, edge_attr, batch, demographics, emb,
           Wrel1, brel1, Wroot1, Wrel2, brel2, Wroot2, Wrel3, brel3, Wroot3,
           Wc1, bc1, Wc2, bc2):
    src = edge_index[0]
    dst = edge_index[1]
    w = edge_attr[:, 0]

    srcp = jnp.pad(src, (0, E_PAD - E))
    dstp = jnp.pad(dst, (0, E_PAD - E))
    wp = jnp.pad(w, (0, E_PAD - E))
    xp = jnp.pad(x, (0, N_PAD - N))
    batchp = jnp.pad(batch, (0, N_PAD - N), constant_values=NG)

    h0 = _sc_gather_rows(emb, xp)

    bh0 = _sc_edge_pass(h0, srcp, dstp, wp)
    bd0 = _sc_d_pass(srcp, dstp, wp, None)
    h1, P1, d, sd, P0, cnt = _tc_combine(
        bh0[0], bh0[1], bd0[0], bd0[1], h0, batchp,
        want_h=True, want_d=True, want_extra=True)

    bh1 = _sc_edge_pass(h1, srcp, dstp, wp)
    bd1 = _sc_d_pass(srcp, dstp, wp, d)
    h2, P2, _d2, sd2, _cnt2 = _tc_combine(
        bh1[0], bh1[1], bd1[0], bd1[1], None, batchp,
        want_h=True, want_d=True, want_extra=False)

    bh2 = _sc_edge_pass(h2, srcp, dstp, wp)
    P3, _cnt3 = _tc_combine(
        bh2[0], bh2[1], None, None, None, batchp,
        want_h=False, want_d=False, want_extra=False)

    return _tc_head(P0, P1, P2, P3, sd, sd2, cnt, demographics,
                    Wrel1, brel1, Wroot1, Wrel2, brel2, Wroot2,
                    Wrel3, brel3, Wroot3,
                    Wc1[:, :64], Wc1[:, 64:], bc1, Wc2, bc2)


# speedup vs baseline: 10.5118x; 1.1512x over previous
import functools

import jax
import jax.numpy as jnp
from jax import lax
from jax.experimental import pallas as pl
from jax.experimental.pallas import tpu as pltpu
from jax.experimental.pallas import tpu_sc as plsc

N = 100000
E = 1600000
NG = 16
F = 16
NC = 2
NS = 16
NW = NC * NS
CH = 256
CPW = 196
EPW = CPW * CH
E_PAD = NW * EPW
N_PAD = 100352
NPS = N_PAD // NS
SB = 14
NSB = CPW // SB
BLK = 2048
GRID = N_PAD // BLK

@functools.cache
def _mesh():
    return plsc.VectorSubcoreMesh(core_axis_name="c", subcore_axis_name="s",
                                  num_cores=NC, num_subcores=NS)


def _sc_gather_rows(table, idx):
    npw = N_PAD // NW
    c_sz = 64
    n_ch = npw // c_sz

    @functools.partial(
        pl.kernel,
        mesh=_mesh(),
        out_type=jax.ShapeDtypeStruct((N_PAD, F), jnp.float32),
        compiler_params=pltpu.CompilerParams(use_tc_tiling_on_sc=False),
        scratch_types=[
            pltpu.VMEM((npw,), jnp.int32),
            pltpu.VMEM((c_sz, F), jnp.float32),
            pltpu.SemaphoreType.DMA,
        ],
    )
    def k(table_hbm, idx_hbm, out_hbm, idx_v, rows_v, sem):
        wid = lax.axis_index("s") * NC + lax.axis_index("c")
        base = wid * npw
        pltpu.sync_copy(idx_hbm.at[pl.ds(base, npw)], idx_v)

        def body(c, carry):
            off = c * c_sz
            pltpu.async_copy(table_hbm.at[idx_v.at[pl.ds(off, c_sz)]], rows_v, sem).wait()
            pltpu.sync_copy(rows_v, out_hbm.at[pl.ds(base + off, c_sz), :])
            return carry

        lax.fori_loop(0, n_ch, body, 0)

    return k(table, idx)


def _sc_edge_pass(hprev, srcp, dstp, wp):
    scratch = [
        pltpu.VMEM((SB * CH,), jnp.int32),
        pltpu.VMEM((SB * CH,), jnp.int32),
        pltpu.VMEM((SB * CH,), jnp.float32),
        pltpu.VMEM((CH, F), jnp.float32),
        pltpu.VMEM((CH, F), jnp.float32),
        pltpu.VMEM((128, F), jnp.float32),
        pltpu.VMEM_SHARED((N_PAD, F), jnp.float32),
        pltpu.SemaphoreType.DMA,
        pltpu.SemaphoreType.DMA,
        pltpu.SemaphoreType.DMA,
        pltpu.SemaphoreType.DMA,
    ]

    @functools.partial(pl.kernel, mesh=_mesh(),
                       out_type=jax.ShapeDtypeStruct((NC, N_PAD, F),
                                                     jnp.float32),
                       compiler_params=pltpu.CompilerParams(
                           use_tc_tiling_on_sc=False),
                       scratch_types=scratch)
    def k(h_hbm, src_hbm, dst_hbm, w_hbm, bh_hbm,
          src_v, dst_v, w_v, rows_a, rows_b, zf_v, acc_sh,
          sga, sgb, ssa, ssb):
        core = lax.axis_index("c")
        sub = lax.axis_index("s")
        wid = sub * NC + core

        zero16 = jnp.zeros((16,), jnp.float32)
        for j in range(128):
            zf_v[j] = zero16
        def zbody(c, carry):
            pltpu.sync_copy(zf_v,
                            acc_sh.at[pl.ds(sub * NPS + c * 128, 128), :])
            return carry
        lax.fori_loop(0, NPS // 128, zbody, 0)
        plsc.subcore_barrier()

        ebase = wid * EPW
        npairs = SB // 2

        def scale(rows_v, woff):
            def gbody(g, carry):
                wv = w_v[pl.ds(woff + g * 16, 16)]
                base = g * 16
                for l in range(16):
                    rows_v[base + l] = rows_v[base + l] * wv[l]
                return carry
            lax.fori_loop(0, CH // 16, gbody, 0, unroll=4)

        def wait_g(sem, rows_v):
            pltpu.make_async_copy(h_hbm.at[pl.ds(0, CH), :], rows_v, sem).wait()

        def wait_s(sem, rows_v, off):
            pltpu.make_async_copy(
                rows_v, acc_sh.at[dst_v.at[pl.ds(off, CH)]], sem).wait()

        def sb_body(s, carry):
            sb0 = ebase + s * (SB * CH)
            pltpu.sync_copy(src_hbm.at[pl.ds(sb0, SB * CH)], src_v)
            pltpu.sync_copy(w_hbm.at[pl.ds(sb0, SB * CH)], w_v)
            pltpu.sync_copy(dst_hbm.at[pl.ds(sb0, SB * CH)], dst_v)
            pltpu.async_copy(h_hbm.at[src_v.at[pl.ds(0, CH)]], rows_a, sga)
            pltpu.async_copy(h_hbm.at[src_v.at[pl.ds(CH, CH)]], rows_b, sgb)

            def pair(t, carry2):
                offa = t * (2 * CH)
                offb = offa + CH
                wait_g(sga, rows_a)
                scale(rows_a, offa)
                pltpu.sync_copy(rows_a, acc_sh.at[dst_v.at[pl.ds(offa, CH)]],
                                add=True)
                wait_g(sgb, rows_b)
                scale(rows_b, offb)
                pltpu.sync_copy(rows_b, acc_sh.at[dst_v.at[pl.ds(offb, CH)]],
                                add=True)

                @pl.when(t < npairs - 1)
                def _next():
                    pltpu.async_copy(
                        h_hbm.at[src_v.at[pl.ds(offa + 2 * CH, CH)]],
                        rows_a, sga)
                    pltpu.async_copy(
                        h_hbm.at[src_v.at[pl.ds(offb + 2 * CH, CH)]],
                        rows_b, sgb)
                return carry2

            lax.fori_loop(0, npairs, pair, 0)
            return carry

        lax.fori_loop(0, NSB, sb_body, 0)

        plsc.subcore_barrier()
        r0 = sub * NPS
        pltpu.sync_copy(acc_sh.at[pl.ds(r0, NPS), :],
                        bh_hbm.at[core, pl.ds(r0, NPS), :])

    return k(hprev, srcp, dstp, wp)


def _sc_d_pass(srcp, dstp, wp, dvec):
    want_d2 = dvec is not None
    scratch = [
        pltpu.VMEM((SB * CH,), jnp.int32),
        pltpu.VMEM((SB * CH,), jnp.int32),
        pltpu.VMEM((SB * CH,), jnp.float32),
        pltpu.VMEM((CH,), jnp.float32),
        pltpu.VMEM((CH,), jnp.float32),
        pltpu.VMEM((SB * CH,), jnp.float32),
        pltpu.VMEM((NPS,), jnp.float32),
        pltpu.VMEM_SHARED((N_PAD,), jnp.float32),
        pltpu.SemaphoreType.DMA,
        pltpu.SemaphoreType.DMA,
        pltpu.SemaphoreType.DMA,
    ]

    @functools.partial(pl.kernel, mesh=_mesh(),
                       out_type=jax.ShapeDtypeStruct((NC, N_PAD), jnp.float32),
                       compiler_params=pltpu.CompilerParams(
                           use_tc_tiling_on_sc=False),
                       scratch_types=scratch)
    def k(src_hbm, dst_hbm, w_hbm, d_hbm, bd_hbm,
          src_v, dst_v, w_v, dsrc_a, dsrc_b, d2_v, zd_v, accd_sh,
          sga, sgb, ssem):
        core = lax.axis_index("c")
        sub = lax.axis_index("s")
        wid = sub * NC + core

        zero16 = jnp.zeros((16,), jnp.float32)
        for j in range(NPS // 16):
            zd_v[pl.ds(j * 16, 16)] = zero16
        pltpu.sync_copy(zd_v, accd_sh.at[pl.ds(sub * NPS, NPS)])
        plsc.subcore_barrier()

        ebase = wid * EPW
        npairs = SB // 2

        def wait_g(sem, buf):
            pltpu.make_async_copy(d_hbm.at[pl.ds(0, CH)], buf, sem).wait()

        def drain_s(n):
            def dbody(c, carry):
                pltpu.make_async_copy(
                    w_v.at[pl.ds(0, CH)],
                    accd_sh.at[dst_v.at[pl.ds(0, CH)]], ssem).wait()
                return carry
            lax.fori_loop(0, n, dbody, 0)

        def sb_body(s, carry):
            sb0 = ebase + s * (SB * CH)
            if want_d2:
                pltpu.sync_copy(src_hbm.at[pl.ds(sb0, SB * CH)], src_v)
            pltpu.sync_copy(w_hbm.at[pl.ds(sb0, SB * CH)], w_v)
            pltpu.sync_copy(dst_hbm.at[pl.ds(sb0, SB * CH)], dst_v)

            if not want_d2:
                def ch_body(c, carry2):
                    off = c * CH
                    pltpu.async_copy(w_v.at[pl.ds(off, CH)],
                                     accd_sh.at[dst_v.at[pl.ds(off, CH)]],
                                     ssem, add=True)
                    return carry2
                lax.fori_loop(0, SB, ch_body, 0)
                drain_s(SB)
            else:
                pltpu.async_copy(d_hbm.at[src_v.at[pl.ds(0, CH)]],
                                 dsrc_a, sga)
                pltpu.async_copy(d_hbm.at[src_v.at[pl.ds(CH, CH)]],
                                 dsrc_b, sgb)

                def pair(t, carry2):
                    offa = t * (2 * CH)
                    offb = offa + CH
                    wait_g(sga, dsrc_a)
                    for g in range(CH // 16):
                        d2_v[pl.ds(offa + g * 16, 16)] = (
                            dsrc_a[pl.ds(g * 16, 16)]
                            * w_v[pl.ds(offa + g * 16, 16)])
                    wait_g(sgb, dsrc_b)
                    for g in range(CH // 16):
                        d2_v[pl.ds(offb + g * 16, 16)] = (
                            dsrc_b[pl.ds(g * 16, 16)]
                            * w_v[pl.ds(offb + g * 16, 16)])

                    @pl.when(t < npairs - 1)
                    def _next():
                        pltpu.async_copy(
                            d_hbm.at[src_v.at[pl.ds(offa + 2 * CH, CH)]],
                            dsrc_a, sga)
                        pltpu.async_copy(
                            d_hbm.at[src_v.at[pl.ds(offb + 2 * CH, CH)]],
                            dsrc_b, sgb)
                    pltpu.async_copy(d2_v.at[pl.ds(offa, CH)],
                                     accd_sh.at[dst_v.at[pl.ds(offa, CH)]],
                                     ssem, add=True)
                    pltpu.async_copy(d2_v.at[pl.ds(offb, CH)],
                                     accd_sh.at[dst_v.at[pl.ds(offb, CH)]],
                                     ssem, add=True)
                    return carry2

                lax.fori_loop(0, npairs, pair, 0)
                drain_s(SB)
            return carry

        lax.fori_loop(0, NSB, sb_body, 0)

        plsc.subcore_barrier()
        r0 = sub * NPS
        pltpu.sync_copy(accd_sh.at[pl.ds(r0, NPS)],
                        bd_hbm.at[core, pl.ds(r0, NPS)])

    return k(srcp, dstp, wp,
             dvec if want_d2 else jnp.zeros((N_PAD,), jnp.float32))


def _tc_combine(bh0, bh1, bd0, bd1, extra, batchp, want_h, want_d, want_extra):
    outs = []
    if want_h:
        outs.append(jax.ShapeDtypeStruct((N_PAD, F), jnp.float32))
    outs.append(jax.ShapeDtypeStruct((NG, F), jnp.float32))
    if want_d:
        outs.append(jax.ShapeDtypeStruct((N_PAD,), jnp.float32))
    if want_d:
        outs.append(jax.ShapeDtypeStruct((NG, 1), jnp.float32))
    if want_extra:
        outs.append(jax.ShapeDtypeStruct((NG, F), jnp.float32))
    outs.append(jax.ShapeDtypeStruct((NG, 1), jnp.float32))

    def body(*refs):
        i = pl.program_id(0)
        it = iter(refs)
        bh0_r = next(it); bh1_r = next(it)
        bd0_r = next(it) if want_d else None
        bd1_r = next(it) if want_d else None
        ex_r = next(it) if want_extra else None
        bt_r = next(it)
        h_r = next(it) if want_h else None
        ph_r = next(it)
        d_r = next(it) if want_d else None
        sd_r = next(it) if want_d else None
        pe_r = next(it) if want_extra else None
        cnt_r = next(it)
        ph_s = next(it)
        sd_s = next(it) if want_d else None
        pe_s = next(it) if want_extra else None
        cnt_s = next(it)

        @pl.when(i == 0)
        def _init():
            ph_s[...] = jnp.zeros((NG, F), jnp.float32)
            cnt_s[...] = jnp.zeros((NG, 1), jnp.float32)
            if want_d:
                sd_s[...] = jnp.zeros((NG, 1), jnp.float32)
            if want_extra:
                pe_s[...] = jnp.zeros((NG, F), jnp.float32)

        hb = bh0_r[...] + bh1_r[...]
        if want_h:
            h_r[...] = hb
        bt = bt_r[...]
        oh = (bt[:, None] == lax.broadcasted_iota(jnp.int32, (1, NG), 1)
              ).astype(jnp.float32)
        dn = (((0,), (0,)), ((), ()))
        ph_s[...] += lax.dot_general(oh, hb, dn,
                                     preferred_element_type=jnp.float32)
        cnt_s[...] += lax.dot_general(oh, jnp.ones((BLK, 1), jnp.float32), dn,
                                      preferred_element_type=jnp.float32)
        if want_d:
            db = bd0_r[...] + bd1_r[...]
            d_r[...] = db
            sd_s[...] += lax.dot_general(oh, db[:, None], dn,
                                         preferred_element_type=jnp.float32)
        if want_extra:
            pe_s[...] += lax.dot_general(oh, ex_r[...], dn,
                                         preferred_element_type=jnp.float32)

        @pl.when(i == GRID - 1)
        def _fin():
            ph_r[...] = ph_s[...]
            cnt_r[...] = cnt_s[...]
            if want_d:
                sd_r[...] = sd_s[...]
            if want_extra:
                pe_r[...] = pe_s[...]

    row = pl.BlockSpec((BLK, F), lambda i: (i, 0))
    vec = pl.BlockSpec((BLK,), lambda i: (i,))
    fixg = pl.BlockSpec((NG, F), lambda i: (0, 0))
    fix1 = pl.BlockSpec((NG, 1), lambda i: (0, 0))

    in_specs = [row, row]
    inputs = [bh0, bh1]
    if want_d:
        in_specs += [vec, vec]
        inputs += [bd0, bd1]
    if want_extra:
        in_specs += [row]
        inputs += [extra]
    in_specs += [vec]
    inputs += [batchp]

    out_specs = []
    if want_h:
        out_specs.append(row)
    out_specs.append(fixg)
    if want_d:
        out_specs += [vec, fix1]
    if want_extra:
        out_specs.append(fixg)
    out_specs.append(fix1)

    scratch = [pltpu.VMEM((NG, F), jnp.float32)]
    if want_d:
        scratch.append(pltpu.VMEM((NG, 1), jnp.float32))
    if want_extra:
        scratch.append(pltpu.VMEM((NG, F), jnp.float32))
    scratch.append(pltpu.VMEM((NG, 1), jnp.float32))

    return pl.pallas_call(
        body,
        grid=(GRID,),
        in_specs=in_specs,
        out_specs=tuple(out_specs),
        out_shape=tuple(outs),
        scratch_shapes=scratch,
    )(*inputs)


def _tc_head(P0, P1, P2, P3, sd, sd2, cnt, demo,
             Wrel1, brel1, Wroot1, Wrel2, brel2, Wroot2, Wrel3, brel3, Wroot3,
             Wc1a, Wc1b, bc1, Wc2, bc2):
    def body(p0, p1, p2, p3, sd_r, sd2_r, cnt_r, demo_r,
             wr1, br1, wo1, wr2, br2, wo2, wr3, br3, wo3,
             wc1a, wc1b, bc1_r, wc2, bc2_r, out_r):
        dnT = (((1,), (1,)), ((), ()))
        mm = lambda a, b: jnp.dot(a, b, preferred_element_type=jnp.float32)
        rmm = lambda a, b: lax.dot_general(a, b, dnT,
                                           preferred_element_type=jnp.float32)
        wr1a, wo1a = wr1[...], wo1[...]
        wr2a, wo2a = wr2[...], wo2[...]
        wr3a, wo3a = wr3[...], wo3[...]
        wo21 = mm(wo2a, wo1a)
        wr2o1 = mm(wr2a, wo1a)
        wo2r1 = mm(wo2a, wr1a)
        wr21 = mm(wr2a, wr1a)
        C0T = mm(wo3a, wo21)
        C1T = mm(wo3a, wr2o1) + mm(wo3a, wo2r1) + mm(wr3a, wo21)
        C2T = mm(wo3a, wr21) + mm(wr3a, wr2o1) + mm(wr3a, wo2r1)
        C3T = mm(wr3a, wr21)
        b1r, b2r, b3r = br1[...], br2[...], br3[...]
        g0r = rmm(rmm(b1r, wo2a), wo3a) + rmm(b2r, wo3a) + b3r
        g1r = rmm(rmm(b1r, wr2a), wo3a) + rmm(rmm(b1r, wo2a), wr3a) \
            + rmm(b2r, wr3a)
        g2r = rmm(rmm(b1r, wr2a), wr3a)

        cnt = cnt_r[...]
        inv = 1.0 / jnp.maximum(cnt, 1.0)
        has = (cnt > 0.0).astype(jnp.float32)
        pooled = (
            rmm(p0[...] * inv, C0T) + rmm(p1[...] * inv, C1T)
            + rmm(p2[...] * inv, C2T) + rmm(p3[...] * inv, C3T)
            + has * g0r + (sd_r[...] * inv) * g1r
            + (sd2_r[...] * inv) * g2r)
        z = (rmm(pooled, wc1a[...]) + rmm(demo_r[...], wc1b[...])
             + bc1_r[...])
        z = jnp.maximum(z, 0.0)
        out_r[...] = rmm(z, wc2[...]) + bc2_r[...]

    return pl.pallas_call(
        body,
        out_shape=jax.ShapeDtypeStruct((NG, 10), jnp.float32),
    )(P0, P1, P2, P3, sd, sd2, cnt, demo,
      Wrel1, brel1[None, :], Wroot1, Wrel2, brel2[None, :], Wroot2,
      Wrel3, brel3[None, :], Wroot3,
      Wc1a, Wc1b, bc1[None, :], Wc2, bc2[None, :])


def kernel(x, edge_index, edge_attr, batch, demographics, emb,
           Wrel1, brel1, Wroot1, Wrel2, brel2, Wroot2, Wrel3, brel3, Wroot3,
           Wc1, bc1, Wc2, bc2):
    src = edge_index[0]
    dst = edge_index[1]
    w = edge_attr[:, 0]

    srcp = jnp.pad(src, (0, E_PAD - E))
    dstp = jnp.pad(dst, (0, E_PAD - E))
    wp = jnp.pad(w, (0, E_PAD - E))
    xp = jnp.pad(x, (0, N_PAD - N))
    batchp = jnp.pad(batch, (0, N_PAD - N), constant_values=NG)

    h0 = _sc_gather_rows(emb, xp)

    bh0 = _sc_edge_pass(h0, srcp, dstp, wp)
    bd0 = _sc_d_pass(srcp, dstp, wp, None)
    h1, P1, d, sd, P0, cnt = _tc_combine(
        bh0[0], bh0[1], bd0[0], bd0[1], h0, batchp,
        want_h=True, want_d=True, want_extra=True)

    bh1 = _sc_edge_pass(h1, srcp, dstp, wp)
    bd1 = _sc_d_pass(srcp, dstp, wp, d)
    h2, P2, _d2, sd2, _cnt2 = _tc_combine(
        bh1[0], bh1[1], bd1[0], bd1[1], None, batchp,
        want_h=True, want_d=True, want_extra=False)

    bh2 = _sc_edge_pass(h2, srcp, dstp, wp)
    P3, _cnt3 = _tc_combine(
        bh2[0], bh2[1], None, None, None, batchp,
        want_h=False, want_d=False, want_extra=False)

    return _tc_head(P0, P1, P2, P3, sd, sd2, cnt, demographics,
                    Wrel1, brel1, Wroot1, Wrel2, brel2, Wroot2,
                    Wrel3, brel3, Wroot3,
                    Wc1[:, :64], Wc1[:, 64:], bc1, Wc2, bc2)


# speedup vs baseline: 11.3998x; 1.0845x over previous
import functools

import jax
import jax.numpy as jnp
from jax import lax
from jax.experimental import pallas as pl
from jax.experimental.pallas import tpu as pltpu
from jax.experimental.pallas import tpu_sc as plsc

N = 100000
E = 1600000
NG = 16
F = 16
NC = 2
NS = 16
NW = NC * NS
CH = 224
CPW = 224
EPW = CPW * CH
E_PAD = NW * EPW
N_PAD = 100352
NPS = N_PAD // NS
SB = 28
NSB = CPW // SB
BLK = 2048
GRID = N_PAD // BLK

@functools.cache
def _mesh():
    return plsc.VectorSubcoreMesh(core_axis_name="c", subcore_axis_name="s",
                                  num_cores=NC, num_subcores=NS)


def _sc_gather_rows(table, idx):
    npw = N_PAD // NW
    c_sz = 64
    n_ch = npw // c_sz

    @functools.partial(
        pl.kernel,
        mesh=_mesh(),
        out_type=jax.ShapeDtypeStruct((N_PAD, F), jnp.float32),
        compiler_params=pltpu.CompilerParams(use_tc_tiling_on_sc=False),
        scratch_types=[
            pltpu.VMEM((npw,), jnp.int32),
            pltpu.VMEM((c_sz, F), jnp.float32),
            pltpu.SemaphoreType.DMA,
        ],
    )
    def k(table_hbm, idx_hbm, out_hbm, idx_v, rows_v, sem):
        wid = lax.axis_index("s") * NC + lax.axis_index("c")
        base = wid * npw
        pltpu.sync_copy(idx_hbm.at[pl.ds(base, npw)], idx_v)

        def body(c, carry):
            off = c * c_sz
            pltpu.async_copy(table_hbm.at[idx_v.at[pl.ds(off, c_sz)]], rows_v, sem).wait()
            pltpu.sync_copy(rows_v, out_hbm.at[pl.ds(base + off, c_sz), :])
            return carry

        lax.fori_loop(0, n_ch, body, 0)

    return k(table, idx)


def _sc_edge_pass(hprev, srcp, dstp, wp):
    scratch = [
        pltpu.VMEM((SB * CH,), jnp.int32),
        pltpu.VMEM((SB * CH,), jnp.int32),
        pltpu.VMEM((SB * CH,), jnp.float32),
        pltpu.VMEM((CH, F), jnp.float32),
        pltpu.VMEM((CH, F), jnp.float32),
        pltpu.VMEM((128, F), jnp.float32),
        pltpu.VMEM_SHARED((N_PAD, F), jnp.float32),
        pltpu.SemaphoreType.DMA,
        pltpu.SemaphoreType.DMA,
        pltpu.SemaphoreType.DMA,
    ]

    @functools.partial(pl.kernel, mesh=_mesh(),
                       out_type=jax.ShapeDtypeStruct((NC, N_PAD, F),
                                                     jnp.float32),
                       compiler_params=pltpu.CompilerParams(
                           use_tc_tiling_on_sc=False),
                       scratch_types=scratch)
    def k(h_hbm, src_hbm, dst_hbm, w_hbm, bh_hbm,
          src_v, dst_v, w_v, rows_a, rows_b, zf_v, acc_sh,
          sga, sgb, ssem):
        core = lax.axis_index("c")
        sub = lax.axis_index("s")
        wid = sub * NC + core

        zero16 = jnp.zeros((16,), jnp.float32)
        for j in range(128):
            zf_v[j] = zero16
        def zbody(c, carry):
            pltpu.sync_copy(zf_v,
                            acc_sh.at[pl.ds(sub * NPS + c * 128, 128), :])
            return carry
        lax.fori_loop(0, NPS // 128, zbody, 0)
        plsc.subcore_barrier()

        ebase = wid * EPW
        npairs = SB // 2
        bufs = [(rows_a, sga), (rows_b, sgb)]

        def scale(rows_v, woff):
            def gbody(g, carry):
                wv = w_v[pl.ds(woff + g * 16, 16)]
                base = g * 16
                for l in range(16):
                    rows_v[base + l] = rows_v[base + l] * wv[l]
                return carry
            lax.fori_loop(0, CH // 16, gbody, 0, unroll=4)

        def wait_g(sem, rows_v):
            pltpu.make_async_copy(h_hbm.at[pl.ds(0, CH), :], rows_v, sem).wait()

        def drain_s(n):
            def dbody(c, carry):
                pltpu.make_async_copy(
                    rows_a, acc_sh.at[dst_v.at[pl.ds(0, CH)]], ssem).wait()
                return carry
            lax.fori_loop(0, n, dbody, 0)

        def sb_body(s, carry):
            sb0 = ebase + s * (SB * CH)
            pltpu.sync_copy(src_hbm.at[pl.ds(sb0, SB * CH)], src_v)
            pltpu.sync_copy(w_hbm.at[pl.ds(sb0, SB * CH)], w_v)
            pltpu.sync_copy(dst_hbm.at[pl.ds(sb0, SB * CH)], dst_v)
            for i, (buf, sem) in enumerate(bufs):
                pltpu.async_copy(h_hbm.at[src_v.at[pl.ds(i * CH, CH)]],
                                 buf, sem)

            def pair(q, carry2):
                off0 = q * (2 * CH)
                for i, (buf, sem) in enumerate(bufs):
                    off = off0 + i * CH
                    wait_g(sem, buf)
                    scale(buf, off)
                    pltpu.async_copy(buf, acc_sh.at[dst_v.at[pl.ds(off, CH)]],
                                     ssem, add=True)
                drain_s(2)

                @pl.when(q < npairs - 1)
                def _next():
                    for i, (buf, sem) in enumerate(bufs):
                        pltpu.async_copy(
                            h_hbm.at[src_v.at[pl.ds(off0 + (2 + i) * CH, CH)]],
                            buf, sem)
                return carry2

            lax.fori_loop(0, npairs, pair, 0)
            return carry

        lax.fori_loop(0, NSB, sb_body, 0)

        plsc.subcore_barrier()
        r0 = sub * NPS
        pltpu.sync_copy(acc_sh.at[pl.ds(r0, NPS), :],
                        bh_hbm.at[core, pl.ds(r0, NPS), :])

    return k(hprev, srcp, dstp, wp)


def _sc_d_pass(srcp, dstp, wp, dvec):
    want_d2 = dvec is not None
    scratch = [
        pltpu.VMEM((SB * CH,), jnp.int32),
        pltpu.VMEM((SB * CH,), jnp.int32),
        pltpu.VMEM((SB * CH,), jnp.float32),
        pltpu.VMEM((CH,), jnp.float32),
        pltpu.VMEM((CH,), jnp.float32),
        pltpu.VMEM((CH,), jnp.float32),
        pltpu.VMEM((CH,), jnp.float32),
        pltpu.VMEM((SB * CH,), jnp.float32),
        pltpu.VMEM((NPS,), jnp.float32),
        pltpu.VMEM_SHARED((N_PAD,), jnp.float32),
        pltpu.SemaphoreType.DMA,
        pltpu.SemaphoreType.DMA,
        pltpu.SemaphoreType.DMA,
        pltpu.SemaphoreType.DMA,
        pltpu.SemaphoreType.DMA,
    ]

    @functools.partial(pl.kernel, mesh=_mesh(),
                       out_type=jax.ShapeDtypeStruct((NC, N_PAD), jnp.float32),
                       compiler_params=pltpu.CompilerParams(
                           use_tc_tiling_on_sc=False),
                       scratch_types=scratch)
    def k(src_hbm, dst_hbm, w_hbm, d_hbm, bd_hbm,
          src_v, dst_v, w_v, dsrc_a, dsrc_b, dsrc_c, dsrc_d, d2_v, zd_v,
          accd_sh, sga, sgb, sgc, sgd, ssem):
        core = lax.axis_index("c")
        sub = lax.axis_index("s")
        wid = sub * NC + core

        zero16 = jnp.zeros((16,), jnp.float32)
        for j in range(NPS // 16):
            zd_v[pl.ds(j * 16, 16)] = zero16
        pltpu.sync_copy(zd_v, accd_sh.at[pl.ds(sub * NPS, NPS)])
        plsc.subcore_barrier()

        ebase = wid * EPW
        nquads = SB // 4
        dbufs = [(dsrc_a, sga), (dsrc_b, sgb), (dsrc_c, sgc), (dsrc_d, sgd)]

        def wait_g(sem, buf):
            pltpu.make_async_copy(d_hbm.at[pl.ds(0, CH)], buf, sem).wait()

        def drain_s(n):
            def dbody(c, carry):
                pltpu.make_async_copy(
                    w_v.at[pl.ds(0, CH)],
                    accd_sh.at[dst_v.at[pl.ds(0, CH)]], ssem).wait()
                return carry
            lax.fori_loop(0, n, dbody, 0)

        def sb_body(s, carry):
            sb0 = ebase + s * (SB * CH)
            if want_d2:
                pltpu.sync_copy(src_hbm.at[pl.ds(sb0, SB * CH)], src_v)
            pltpu.sync_copy(w_hbm.at[pl.ds(sb0, SB * CH)], w_v)
            pltpu.sync_copy(dst_hbm.at[pl.ds(sb0, SB * CH)], dst_v)

            if not want_d2:
                def ch_body(c, carry2):
                    off = c * CH
                    pltpu.async_copy(w_v.at[pl.ds(off, CH)],
                                     accd_sh.at[dst_v.at[pl.ds(off, CH)]],
                                     ssem, add=True)
                    return carry2
                lax.fori_loop(0, SB, ch_body, 0)
                drain_s(SB)
            else:
                for i, (buf, sem) in enumerate(dbufs):
                    pltpu.async_copy(d_hbm.at[src_v.at[pl.ds(i * CH, CH)]],
                                     buf, sem)

                def quad(q, carry2):
                    off0 = q * (4 * CH)
                    for i, (buf, sem) in enumerate(dbufs):
                        off = off0 + i * CH
                        wait_g(sem, buf)
                        for g in range(CH // 16):
                            d2_v[pl.ds(off + g * 16, 16)] = (
                                buf[pl.ds(g * 16, 16)]
                                * w_v[pl.ds(off + g * 16, 16)])
                        pltpu.async_copy(
                            d2_v.at[pl.ds(off, CH)],
                            accd_sh.at[dst_v.at[pl.ds(off, CH)]],
                            ssem, add=True)

                    @pl.when(q < nquads - 1)
                    def _next():
                        for i, (buf, sem) in enumerate(dbufs):
                            pltpu.async_copy(
                                d_hbm.at[src_v.at[pl.ds(off0 + (4 + i) * CH,
                                                        CH)]],
                                buf, sem)
                    return carry2

                lax.fori_loop(0, nquads, quad, 0)
                drain_s(SB)
            return carry

        lax.fori_loop(0, NSB, sb_body, 0)

        plsc.subcore_barrier()
        r0 = sub * NPS
        pltpu.sync_copy(accd_sh.at[pl.ds(r0, NPS)],
                        bd_hbm.at[core, pl.ds(r0, NPS)])

    return k(srcp, dstp, wp,
             dvec if want_d2 else jnp.zeros((N_PAD,), jnp.float32))


def _tc_combine(bh0, bh1, bd0, bd1, extra, batchp, want_h, want_d, want_extra):
    outs = []
    if want_h:
        outs.append(jax.ShapeDtypeStruct((N_PAD, F), jnp.float32))
    outs.append(jax.ShapeDtypeStruct((NG, F), jnp.float32))
    if want_d:
        outs.append(jax.ShapeDtypeStruct((N_PAD,), jnp.float32))
    if want_d:
        outs.append(jax.ShapeDtypeStruct((NG, 1), jnp.float32))
    if want_extra:
        outs.append(jax.ShapeDtypeStruct((NG, F), jnp.float32))
    outs.append(jax.ShapeDtypeStruct((NG, 1), jnp.float32))

    def body(*refs):
        i = pl.program_id(0)
        it = iter(refs)
        bh0_r = next(it); bh1_r = next(it)
        bd0_r = next(it) if want_d else None
        bd1_r = next(it) if want_d else None
        ex_r = next(it) if want_extra else None
        bt_r = next(it)
        h_r = next(it) if want_h else None
        ph_r = next(it)
        d_r = next(it) if want_d else None
        sd_r = next(it) if want_d else None
        pe_r = next(it) if want_extra else None
        cnt_r = next(it)
        ph_s = next(it)
        sd_s = next(it) if want_d else None
        pe_s = next(it) if want_extra else None
        cnt_s = next(it)

        @pl.when(i == 0)
        def _init():
            ph_s[...] = jnp.zeros((NG, F), jnp.float32)
            cnt_s[...] = jnp.zeros((NG, 1), jnp.float32)
            if want_d:
                sd_s[...] = jnp.zeros((NG, 1), jnp.float32)
            if want_extra:
                pe_s[...] = jnp.zeros((NG, F), jnp.float32)

        hb = bh0_r[...] + bh1_r[...]
        if want_h:
            h_r[...] = hb
        bt = bt_r[...]
        oh = (bt[:, None] == lax.broadcasted_iota(jnp.int32, (1, NG), 1)
              ).astype(jnp.float32)
        dn = (((0,), (0,)), ((), ()))
        ph_s[...] += lax.dot_general(oh, hb, dn,
                                     preferred_element_type=jnp.float32)
        cnt_s[...] += lax.dot_general(oh, jnp.ones((BLK, 1), jnp.float32), dn,
                                      preferred_element_type=jnp.float32)
        if want_d:
            db = bd0_r[...] + bd1_r[...]
            d_r[...] = db
            sd_s[...] += lax.dot_general(oh, db[:, None], dn,
                                         preferred_element_type=jnp.float32)
        if want_extra:
            pe_s[...] += lax.dot_general(oh, ex_r[...], dn,
                                         preferred_element_type=jnp.float32)

        @pl.when(i == GRID - 1)
        def _fin():
            ph_r[...] = ph_s[...]
            cnt_r[...] = cnt_s[...]
            if want_d:
                sd_r[...] = sd_s[...]
            if want_extra:
                pe_r[...] = pe_s[...]

    row = pl.BlockSpec((BLK, F), lambda i: (i, 0))
    vec = pl.BlockSpec((BLK,), lambda i: (i,))
    fixg = pl.BlockSpec((NG, F), lambda i: (0, 0))
    fix1 = pl.BlockSpec((NG, 1), lambda i: (0, 0))

    in_specs = [row, row]
    inputs = [bh0, bh1]
    if want_d:
        in_specs += [vec, vec]
        inputs += [bd0, bd1]
    if want_extra:
        in_specs += [row]
        inputs += [extra]
    in_specs += [vec]
    inputs += [batchp]

    out_specs = []
    if want_h:
        out_specs.append(row)
    out_specs.append(fixg)
    if want_d:
        out_specs += [vec, fix1]
    if want_extra:
        out_specs.append(fixg)
    out_specs.append(fix1)

    scratch = [pltpu.VMEM((NG, F), jnp.float32)]
    if want_d:
        scratch.append(pltpu.VMEM((NG, 1), jnp.float32))
    if want_extra:
        scratch.append(pltpu.VMEM((NG, F), jnp.float32))
    scratch.append(pltpu.VMEM((NG, 1), jnp.float32))

    return pl.pallas_call(
        body,
        grid=(GRID,),
        in_specs=in_specs,
        out_specs=tuple(out_specs),
        out_shape=tuple(outs),
        scratch_shapes=scratch,
    )(*inputs)


def _tc_head(P0, P1, P2, P3, sd, sd2, cnt, demo,
             Wrel1, brel1, Wroot1, Wrel2, brel2, Wroot2, Wrel3, brel3, Wroot3,
             Wc1a, Wc1b, bc1, Wc2, bc2):
    def body(p0, p1, p2, p3, sd_r, sd2_r, cnt_r, demo_r,
             wr1, br1, wo1, wr2, br2, wo2, wr3, br3, wo3,
             wc1a, wc1b, bc1_r, wc2, bc2_r, out_r):
        dnT = (((1,), (1,)), ((), ()))
        mm = lambda a, b: jnp.dot(a, b, preferred_element_type=jnp.float32)
        rmm = lambda a, b: lax.dot_general(a, b, dnT,
                                           preferred_element_type=jnp.float32)
        wr1a, wo1a = wr1[...], wo1[...]
        wr2a, wo2a = wr2[...], wo2[...]
        wr3a, wo3a = wr3[...], wo3[...]
        wo21 = mm(wo2a, wo1a)
        wr2o1 = mm(wr2a, wo1a)
        wo2r1 = mm(wo2a, wr1a)
        wr21 = mm(wr2a, wr1a)
        C0T = mm(wo3a, wo21)
        C1T = mm(wo3a, wr2o1) + mm(wo3a, wo2r1) + mm(wr3a, wo21)
        C2T = mm(wo3a, wr21) + mm(wr3a, wr2o1) + mm(wr3a, wo2r1)
        C3T = mm(wr3a, wr21)
        b1r, b2r, b3r = br1[...], br2[...], br3[...]
        g0r = rmm(rmm(b1r, wo2a), wo3a) + rmm(b2r, wo3a) + b3r
        g1r = rmm(rmm(b1r, wr2a), wo3a) + rmm(rmm(b1r, wo2a), wr3a) \
            + rmm(b2r, wr3a)
        g2r = rmm(rmm(b1r, wr2a), wr3a)

        cnt = cnt_r[...]
        inv = 1.0 / jnp.maximum(cnt, 1.0)
        has = (cnt > 0.0).astype(jnp.float32)
        pooled = (
            rmm(p0[...] * inv, C0T) + rmm(p1[...] * inv, C1T)
            + rmm(p2[...] * inv, C2T) + rmm(p3[...] * inv, C3T)
            + has * g0r + (sd_r[...] * inv) * g1r
            + (sd2_r[...] * inv) * g2r)
        z = (rmm(pooled, wc1a[...]) + rmm(demo_r[...], wc1b[...])
             + bc1_r[...])
        z = jnp.maximum(z, 0.0)
        out_r[...] = rmm(z, wc2[...]) + bc2_r[...]

    return pl.pallas_call(
        body,
        out_shape=jax.ShapeDtypeStruct((NG, 10), jnp.float32),
    )(P0, P1, P2, P3, sd, sd2, cnt, demo,
      Wrel1, brel1[None, :], Wroot1, Wrel2, brel2[None, :], Wroot2,
      Wrel3, brel3[None, :], Wroot3,
      Wc1a, Wc1b, bc1[None, :], Wc2, bc2[None, :])


def kernel(x, edge_index, edge_attr, batch, demographics, emb,
           Wrel1, brel1, Wroot1, Wrel2, brel2, Wroot2, Wrel3, brel3, Wroot3,
           Wc1, bc1, Wc2, bc2):
    src = edge_index[0]
    dst = edge_index[1]
    w = edge_attr[:, 0]

    srcp = jnp.pad(src, (0, E_PAD - E))
    dstp = jnp.pad(dst, (0, E_PAD - E))
    wp = jnp.pad(w, (0, E_PAD - E))
    xp = jnp.pad(x, (0, N_PAD - N))
    batchp = jnp.pad(batch, (0, N_PAD - N), constant_values=NG)

    h0 = _sc_gather_rows(emb, xp)

    bh0 = _sc_edge_pass(h0, srcp, dstp, wp)
    bd0 = _sc_d_pass(srcp, dstp, wp, None)
    h1, P1, d, sd, P0, cnt = _tc_combine(
        bh0[0], bh0[1], bd0[0], bd0[1], h0, batchp,
        want_h=True, want_d=True, want_extra=True)

    bh1 = _sc_edge_pass(h1, srcp, dstp, wp)
    bd1 = _sc_d_pass(srcp, dstp, wp, d)
    h2, P2, _d2, sd2, _cnt2 = _tc_combine(
        bh1[0], bh1[1], bd1[0], bd1[1], None, batchp,
        want_h=True, want_d=True, want_extra=False)

    bh2 = _sc_edge_pass(h2, srcp, dstp, wp)
    P3, _cnt3 = _tc_combine(
        bh2[0], bh2[1], None, None, None, batchp,
        want_h=False, want_d=False, want_extra=False)

    return _tc_head(P0, P1, P2, P3, sd, sd2, cnt, demographics,
                    Wrel1, brel1, Wroot1, Wrel2, brel2, Wroot2,
                    Wrel3, brel3, Wroot3,
                    Wc1[:, :64], Wc1[:, 64:], bc1, Wc2, bc2)
